# baseline (speedup 1.0000x reference)
import functools

import numpy as np
import jax
import jax.numpy as jnp
from jax.experimental import pallas as pl
from jax.experimental.pallas import tpu as pltpu

WB_SEL = 64
WB_NUM = 2


def _lstm_step(x_t, h, c, wih, whh, b_ih, b_hh, H):
    g = (
        jnp.dot(x_t, wih, preferred_element_type=jnp.float32)
        + jnp.dot(h, whh, preferred_element_type=jnp.float32)
        + b_ih
        + b_hh
    )
    i = jax.nn.sigmoid(g[:, 0:H])
    f = jax.nn.sigmoid(g[:, H : 2 * H])
    gg = jnp.tanh(g[:, 2 * H : 3 * H])
    o = jax.nn.sigmoid(g[:, 3 * H : 4 * H])
    c_new = f * c + i * gg
    h_new = o * jnp.tanh(c_new)
    return h_new, c_new


def _fused_kernel(
    x_ref,
    wih0_ref,
    whh0_ref,
    bih0_ref,
    bhh0_ref,
    wih1_ref,
    whh1_ref,
    bih1_ref,
    bhh1_ref,
    y_ref,
    sdw_ref,
    sdb_ref,
    pair_ref,
    expand_ref,
    tri_ref,
    mw1_ref,
    mb1_ref,
    mw2_ref,
    mb2_ref,
    mdw_ref,
    mdb_ref,
    aw_ref,
    ab_ref,
    fw1_ref,
    fb1_ref,
    fw2_ref,
    fb2_ref,
    fdw_ref,
    fdb_ref,
    out_sub_ref,
    out_mm_ref,
    output_ref,
    h0_ref,
    c0_ref,
    h1_ref,
    c1_ref,
    *,
    H,
    L,
):
    t = pl.program_id(0)

    @pl.when(t == 0)
    def _init():
        h0_ref[...] = jnp.zeros_like(h0_ref)
        c0_ref[...] = jnp.zeros_like(c0_ref)
        h1_ref[...] = jnp.zeros_like(h1_ref)
        c1_ref[...] = jnp.zeros_like(c1_ref)

    x_t = x_ref[0]
    h0, c0 = _lstm_step(x_t, h0_ref[...], c0_ref[...], wih0_ref[...], whh0_ref[...], bih0_ref[...], bhh0_ref[...], H)
    h0_ref[...] = h0
    c0_ref[...] = c0
    h1, c1 = _lstm_step(h0, h1_ref[...], c1_ref[...], wih1_ref[...], whh1_ref[...], bih1_ref[...], bhh1_ref[...], H)
    h1_ref[...] = h1
    c1_ref[...] = c1

    @pl.when(t == L - 1)
    def _tail():
        out_sub = jnp.dot(h1, sdw_ref[...], preferred_element_type=jnp.float32) + sdb_ref[...]
        out_sub_ref[...] = out_sub

        p = jax.lax.dot_general(
            out_sub,
            pair_ref[...],
            (((1,), (0,)), ((), ())),
            precision=jax.lax.Precision.HIGHEST,
            preferred_element_type=jnp.float32,
        )

        s = pltpu.bitcast(p, jnp.int32)
        key = jnp.where(s >= 0, s, s ^ jnp.int32(0x7FFFFFFF))

        def body(b, ts):
            bit = jnp.left_shift(jnp.int32(1), jnp.int32(31) - b)
            trial = ts + bit
            cnt = jnp.sum((key >= trial).astype(jnp.int32), axis=1, keepdims=True)
            return jnp.where(cnt >= WB_SEL, trial, ts)

        ts0 = jnp.full((p.shape[0], 1), jnp.int32(-2147483648), jnp.int32)
        ts = jax.lax.fori_loop(0, 32, body, ts0)

        gt = key > ts
        tie = key == ts
        tie_f = tie.astype(jnp.float32)
        prefix = jnp.dot(tie_f, tri_ref[...], preferred_element_type=jnp.float32)
        need = (WB_SEL - jnp.sum(gt.astype(jnp.int32), axis=1, keepdims=True)).astype(jnp.float32)
        maskf = gt.astype(jnp.float32) + tie_f * (prefix < need).astype(jnp.float32)

        mask_wide = jnp.dot(maskf, expand_ref[...], preferred_element_type=jnp.float32)
        input_mm = y_ref[...] * mask_wide

        bf = jnp.bfloat16
        hm = jnp.maximum(jnp.dot(input_mm.astype(bf), mw1_ref[...].astype(bf), preferred_element_type=jnp.float32) + mb1_ref[...], 0.0)
        hm = jnp.maximum(jnp.dot(hm.astype(bf), mw2_ref[...].astype(bf), preferred_element_type=jnp.float32) + mb2_ref[...], 0.0)
        out_mm = jnp.dot(hm.astype(bf), mdw_ref[...].astype(bf), preferred_element_type=jnp.float32) + mdb_ref[...]
        out_mm_ref[...] = out_mm

        cat = jnp.concatenate([out_sub, out_mm], axis=1)
        att = jax.nn.sigmoid(jnp.dot(cat.astype(bf), aw_ref[...].astype(bf), preferred_element_type=jnp.float32) + ab_ref[...])
        fused = cat * att
        hf = jnp.maximum(jnp.dot(fused.astype(bf), fw1_ref[...].astype(bf), preferred_element_type=jnp.float32) + fb1_ref[...], 0.0)
        hf = jnp.maximum(jnp.dot(hf.astype(bf), fw2_ref[...].astype(bf), preferred_element_type=jnp.float32) + fb2_ref[...], 0.0)
        output_ref[...] = jnp.dot(hf.astype(bf), fdw_ref[...].astype(bf), preferred_element_type=jnp.float32) + fdb_ref[...]


def kernel(x, y, lW_ih0, lW_hh0, lb_ih0, lb_hh0, lW_ih1, lW_hh1, lb_ih1, lb_hh1, sdW, sdb, mW1, mb1, mW2, mb2, mdW, mdb, aW, ab, fW1, fb1, fW2, fb2, fdW, fdb):
    x = x.astype(jnp.float32)
    y = y.astype(jnp.float32)
    B, L, Fd = x.shape
    H = lW_hh0.shape[1]
    C = sdW.shape[0]
    G = C // WB_NUM
    H2 = mW1.shape[0]
    H3 = fW1.shape[0]

    xT = jnp.transpose(x, (1, 0, 2))

    gi = np.arange(C) // WB_NUM
    pair = (gi[:, None] == np.arange(G)[None, :]).astype(np.float32)
    tri = (np.arange(G)[:, None] < np.arange(G)[None, :]).astype(np.float32)

    full = lambda shape: pl.BlockSpec(shape, lambda t: tuple(0 for _ in shape))
    row = lambda w: full((1, w))

    out_sub, out_mm, output = pl.pallas_call(
        functools.partial(_fused_kernel, H=H, L=L),
        grid=(L,),
        in_specs=[
            pl.BlockSpec((1, B, Fd), lambda t: (t, 0, 0)),
            full((Fd, 4 * H)),
            full((H, 4 * H)),
            row(4 * H),
            row(4 * H),
            full((H, 4 * H)),
            full((H, 4 * H)),
            row(4 * H),
            row(4 * H),
            full((B, C)),
            full((H, C)),
            row(C),
            full((C, G)),
            full((G, C)),
            full((G, G)),
            full((C, H2)),
            row(H2),
            full((H2, H2)),
            row(H2),
            full((H2, C)),
            row(C),
            full((2 * C, 2 * C)),
            row(2 * C),
            full((2 * C, H3)),
            row(H3),
            full((H3, H3)),
            row(H3),
            full((H3, C)),
            row(C),
        ],
        out_specs=[
            full((B, C)),
            full((B, C)),
            full((B, C)),
        ],
        out_shape=[
            jax.ShapeDtypeStruct((B, C), jnp.float32),
            jax.ShapeDtypeStruct((B, C), jnp.float32),
            jax.ShapeDtypeStruct((B, C), jnp.float32),
        ],
        scratch_shapes=[
            pltpu.VMEM((B, H), jnp.float32),
            pltpu.VMEM((B, H), jnp.float32),
            pltpu.VMEM((B, H), jnp.float32),
            pltpu.VMEM((B, H), jnp.float32),
        ],
    )(
        xT,
        lW_ih0.T,
        lW_hh0.T,
        lb_ih0.reshape(1, -1),
        lb_hh0.reshape(1, -1),
        lW_ih1.T,
        lW_hh1.T,
        lb_ih1.reshape(1, -1),
        lb_hh1.reshape(1, -1),
        y,
        sdW.T,
        sdb.reshape(1, -1),
        jnp.asarray(pair),
        jnp.asarray(pair.T),
        jnp.asarray(tri),
        mW1.T,
        mb1.reshape(1, -1),
        mW2.T,
        mb2.reshape(1, -1),
        mdW.T,
        mdb.reshape(1, -1),
        aW.T,
        ab.reshape(1, -1),
        fW1.T,
        fb1.reshape(1, -1),
        fW2.T,
        fb2.reshape(1, -1),
        fdW.T,
        fdb.reshape(1, -1),
    )
    return (out_sub, out_mm, output)


# speedup vs baseline: 1.1135x; 1.1135x over previous
import functools

import numpy as np
import jax
import jax.numpy as jnp
from jax.experimental import pallas as pl
from jax.experimental.pallas import tpu as pltpu

WB_SEL = 64
WB_NUM = 2


def _lstm_step(x_t, h, c, wih, whh, b_ih, b_hh, H):
    g = (
        jnp.dot(x_t, wih, preferred_element_type=jnp.float32)
        + jnp.dot(h, whh, preferred_element_type=jnp.float32)
        + b_ih
        + b_hh
    )
    i = jax.nn.sigmoid(g[:, 0:H])
    f = jax.nn.sigmoid(g[:, H : 2 * H])
    gg = jnp.tanh(g[:, 2 * H : 3 * H])
    o = jax.nn.sigmoid(g[:, 3 * H : 4 * H])
    c_new = f * c + i * gg
    h_new = o * jnp.tanh(c_new)
    return h_new, c_new


def _fused_kernel(
    x_ref,
    wih0_ref,
    whh0_ref,
    bih0_ref,
    bhh0_ref,
    wih1_ref,
    whh1_ref,
    bih1_ref,
    bhh1_ref,
    y_ref,
    sdw_ref,
    sdb_ref,
    pair_ref,
    expand_ref,
    tri_ref,
    mw1_ref,
    mb1_ref,
    mw2_ref,
    mb2_ref,
    mdw_ref,
    mdb_ref,
    aw_ref,
    ab_ref,
    fw1_ref,
    fb1_ref,
    fw2_ref,
    fb2_ref,
    fdw_ref,
    fdb_ref,
    out_sub_ref,
    out_mm_ref,
    output_ref,
    h0_ref,
    c0_ref,
    h1_ref,
    c1_ref,
    *,
    H,
    L,
    U,
):
    t = pl.program_id(0)

    @pl.when(t == 0)
    def _init():
        h0_ref[...] = jnp.zeros_like(h0_ref)
        c0_ref[...] = jnp.zeros_like(c0_ref)
        h1_ref[...] = jnp.zeros_like(h1_ref)
        c1_ref[...] = jnp.zeros_like(c1_ref)

    h0, c0 = h0_ref[...], c0_ref[...]
    h1, c1 = h1_ref[...], c1_ref[...]
    for u in range(U):
        h0, c0 = _lstm_step(x_ref[u], h0, c0, wih0_ref[...], whh0_ref[...], bih0_ref[...], bhh0_ref[...], H)
        h1, c1 = _lstm_step(h0, h1, c1, wih1_ref[...], whh1_ref[...], bih1_ref[...], bhh1_ref[...], H)
    h0_ref[...] = h0
    c0_ref[...] = c0
    h1_ref[...] = h1
    c1_ref[...] = c1

    @pl.when(t == L // U - 1)
    def _tail():
        out_sub = jnp.dot(h1, sdw_ref[...], preferred_element_type=jnp.float32) + sdb_ref[...]
        out_sub_ref[...] = out_sub

        p = jax.lax.dot_general(
            out_sub,
            pair_ref[...],
            (((1,), (0,)), ((), ())),
            precision=jax.lax.Precision.HIGHEST,
            preferred_element_type=jnp.float32,
        )

        s = pltpu.bitcast(p, jnp.int32)
        key = jnp.where(s >= 0, s, s ^ jnp.int32(0x7FFFFFFF))

        def body(b, ts):
            bit = jnp.left_shift(jnp.int32(1), jnp.int32(31) - b)
            trial = ts + bit
            cnt = jnp.sum((key >= trial).astype(jnp.int32), axis=1, keepdims=True)
            return jnp.where(cnt >= WB_SEL, trial, ts)

        ts0 = jnp.full((p.shape[0], 1), jnp.int32(-2147483648), jnp.int32)
        ts = jax.lax.fori_loop(0, 32, body, ts0)

        gt = key > ts
        tie = key == ts
        tie_f = tie.astype(jnp.float32)
        prefix = jnp.dot(tie_f, tri_ref[...], preferred_element_type=jnp.float32)
        need = (WB_SEL - jnp.sum(gt.astype(jnp.int32), axis=1, keepdims=True)).astype(jnp.float32)
        maskf = gt.astype(jnp.float32) + tie_f * (prefix < need).astype(jnp.float32)

        mask_wide = jnp.dot(maskf, expand_ref[...], preferred_element_type=jnp.float32)
        input_mm = y_ref[...] * mask_wide

        hm = jnp.maximum(jnp.dot(input_mm, mw1_ref[...], preferred_element_type=jnp.float32) + mb1_ref[...], 0.0)
        hm = jnp.maximum(jnp.dot(hm, mw2_ref[...], preferred_element_type=jnp.float32) + mb2_ref[...], 0.0)
        out_mm = jnp.dot(hm, mdw_ref[...], preferred_element_type=jnp.float32) + mdb_ref[...]
        out_mm_ref[...] = out_mm

        cat = jnp.concatenate([out_sub, out_mm], axis=1)
        att = jax.nn.sigmoid(jnp.dot(cat, aw_ref[...], preferred_element_type=jnp.float32) + ab_ref[...])
        fused = cat * att
        hf = jnp.maximum(jnp.dot(fused, fw1_ref[...], preferred_element_type=jnp.float32) + fb1_ref[...], 0.0)
        hf = jnp.maximum(jnp.dot(hf, fw2_ref[...], preferred_element_type=jnp.float32) + fb2_ref[...], 0.0)
        output_ref[...] = jnp.dot(hf, fdw_ref[...], preferred_element_type=jnp.float32) + fdb_ref[...]


def kernel(x, y, lW_ih0, lW_hh0, lb_ih0, lb_hh0, lW_ih1, lW_hh1, lb_ih1, lb_hh1, sdW, sdb, mW1, mb1, mW2, mb2, mdW, mdb, aW, ab, fW1, fb1, fW2, fb2, fdW, fdb):
    x = x.astype(jnp.float32)
    y = y.astype(jnp.float32)
    B, L, Fd = x.shape
    H = lW_hh0.shape[1]
    C = sdW.shape[0]
    G = C // WB_NUM
    H2 = mW1.shape[0]
    H3 = fW1.shape[0]

    xT = jnp.transpose(x, (1, 0, 2))

    gi = np.arange(C) // WB_NUM
    pair = (gi[:, None] == np.arange(G)[None, :]).astype(np.float32)
    tri = (np.arange(G)[:, None] < np.arange(G)[None, :]).astype(np.float32)

    full = lambda shape: pl.BlockSpec(shape, lambda t: tuple(0 for _ in shape))
    row = lambda w: full((1, w))

    U = 2
    assert L % U == 0
    out_sub, out_mm, output = pl.pallas_call(
        functools.partial(_fused_kernel, H=H, L=L, U=U),
        grid=(L // U,),
        in_specs=[
            pl.BlockSpec((U, B, Fd), lambda t: (t, 0, 0)),
            full((Fd, 4 * H)),
            full((H, 4 * H)),
            row(4 * H),
            row(4 * H),
            full((H, 4 * H)),
            full((H, 4 * H)),
            row(4 * H),
            row(4 * H),
            full((B, C)),
            full((H, C)),
            row(C),
            full((C, G)),
            full((G, C)),
            full((G, G)),
            full((C, H2)),
            row(H2),
            full((H2, H2)),
            row(H2),
            full((H2, C)),
            row(C),
            full((2 * C, 2 * C)),
            row(2 * C),
            full((2 * C, H3)),
            row(H3),
            full((H3, H3)),
            row(H3),
            full((H3, C)),
            row(C),
        ],
        out_specs=[
            full((B, C)),
            full((B, C)),
            full((B, C)),
        ],
        out_shape=[
            jax.ShapeDtypeStruct((B, C), jnp.float32),
            jax.ShapeDtypeStruct((B, C), jnp.float32),
            jax.ShapeDtypeStruct((B, C), jnp.float32),
        ],
        scratch_shapes=[
            pltpu.VMEM((B, H), jnp.float32),
            pltpu.VMEM((B, H), jnp.float32),
            pltpu.VMEM((B, H), jnp.float32),
            pltpu.VMEM((B, H), jnp.float32),
        ],
    )(
        xT,
        lW_ih0.T,
        lW_hh0.T,
        lb_ih0.reshape(1, -1),
        lb_hh0.reshape(1, -1),
        lW_ih1.T,
        lW_hh1.T,
        lb_ih1.reshape(1, -1),
        lb_hh1.reshape(1, -1),
        y,
        sdW.T,
        sdb.reshape(1, -1),
        jnp.asarray(pair),
        jnp.asarray(pair.T),
        jnp.asarray(tri),
        mW1.T,
        mb1.reshape(1, -1),
        mW2.T,
        mb2.reshape(1, -1),
        mdW.T,
        mdb.reshape(1, -1),
        aW.T,
        ab.reshape(1, -1),
        fW1.T,
        fb1.reshape(1, -1),
        fW2.T,
        fb2.reshape(1, -1),
        fdW.T,
        fdb.reshape(1, -1),
    )
    return (out_sub, out_mm, output)
